# Initial kernel scaffold; baseline (speedup 1.0000x reference)
import jax
import jax.numpy as jnp
from jax.experimental import pallas as pl


def kernel(x_original, edge_index_original, x_pp, edge_index_pp, x_fm, edge_index_fm, W_pos_nbr, W_pos_self, b_pos, W_neg_nbr, W_neg_self, b_neg, W1, b1, W2, b2, W3, b3):
    raise NotImplementedError("write your pallas kernel here")


# speedup vs baseline: 5.8178x; 5.8178x over previous
import functools

import jax
import jax.numpy as jnp
from jax import lax
from jax.experimental import pallas as pl
from jax.experimental.pallas import tpu as pltpu
from jax.experimental.pallas import tpu_sc as plsc

NC = 2
NS = 16
C = 128


def _sc_aggregate(x_list, src_list, dst_list, n_pad):
    n, d = x_list[0].shape
    e = src_list[0].shape[0]
    num_chunks = e // C
    per_core = num_chunks // NC
    rows_per_tile = n_pad // NS
    n_row_copies = rows_per_tile // C

    mesh = plsc.VectorSubcoreMesh(
        core_axis_name="c", subcore_axis_name="s", num_cores=NC, num_subcores=NS)

    @functools.partial(
        pl.kernel,
        out_type=(
            jax.ShapeDtypeStruct((3, NC, n_pad, d), jnp.float32),
            jax.ShapeDtypeStruct((3, NC, n_pad), jnp.float32),
        ),
        mesh=mesh,
        scratch_types=dict(
            src_v=pltpu.VMEM((C,), jnp.int32),
            dst_v=pltpu.VMEM((C,), jnp.int32),
            rows_v=pltpu.VMEM((C, d), jnp.float32),
            ones_v=pltpu.VMEM((C,), jnp.float32),
            zer_v=pltpu.VMEM((C, d), jnp.float32),
            zer1_v=pltpu.VMEM((rows_per_tile,), jnp.float32),
            acc_sh=pltpu.VMEM_SHARED((n_pad, d), jnp.float32),
            deg_sh=pltpu.VMEM_SHARED((n_pad,), jnp.float32),
            sem=pltpu.SemaphoreType.DMA,
        ),
    )
    def agg_kernel(x0, x1, x2, s0, s1, s2, d0, d1, d2, agg_out, deg_out,
                   src_v, dst_v, rows_v, ones_v, zer_v, zer1_v, acc_sh, deg_sh, sem):
        cid = lax.axis_index("c")
        sid = lax.axis_index("s")
        row_base = sid * rows_per_tile

        def fill_row(r, _):
            for k in range(d // 16):
                zer_v[r, pl.ds(k * 16, 16)] = jnp.zeros((16,), jnp.float32)
            return 0
        lax.fori_loop(0, C, fill_row, 0)
        for k in range(C // 16):
            ones_v[pl.ds(k * 16, 16)] = jnp.ones((16,), jnp.float32)
        for k in range(rows_per_tile // 16):
            zer1_v[pl.ds(k * 16, 16)] = jnp.zeros((16,), jnp.float32)

        rem = per_core % NS
        nchunks = jnp.where(sid < rem, per_core // NS + 1, per_core // NS)

        for g, (x_hbm, src_hbm, dst_hbm) in enumerate(
                ((x0, s0, d0), (x1, s1, d1), (x2, s2, d2))):
            for t in range(n_row_copies):
                pltpu.sync_copy(zer_v, acc_sh.at[pl.ds(row_base + t * C, C)])
            pltpu.sync_copy(zer1_v, deg_sh.at[pl.ds(row_base, rows_per_tile)])
            plsc.subcore_barrier()

            base_chunk = cid * per_core

            def body(i, _):
                off = (base_chunk + sid + i * NS) * C
                pltpu.sync_copy(src_hbm.at[pl.ds(off, C)], src_v)
                pltpu.sync_copy(dst_hbm.at[pl.ds(off, C)], dst_v)
                pltpu.async_copy(x_hbm.at[src_v], rows_v, sem).wait()
                pltpu.sync_copy(rows_v, acc_sh.at[dst_v], add=True)
                pltpu.sync_copy(ones_v, deg_sh.at[dst_v], add=True)
                return 0
            lax.fori_loop(0, nchunks, body, 0)
            plsc.subcore_barrier()

            for t in range(n_row_copies):
                pltpu.sync_copy(acc_sh.at[pl.ds(row_base + t * C, C)],
                                agg_out.at[g, cid, pl.ds(row_base + t * C, C)])
            pltpu.sync_copy(deg_sh.at[pl.ds(row_base, rows_per_tile)],
                            deg_out.at[g, cid, pl.ds(row_base, rows_per_tile)])
            plsc.subcore_barrier()

    return agg_kernel(*x_list, *src_list, *dst_list)


def _dense_body(x3_ref, agg_ref, deg_ref,
                wpn_ref, wps_ref, bpos_ref, wnn_ref, wns_ref, bneg_ref,
                w1_ref, b1_ref, w2_ref, b2_ref, w3_ref, b3_ref,
                o0_ref, o1_ref, o2_ref, o3_ref):
    f32 = jnp.float32
    x3 = x3_ref[...]
    agg = agg_ref[...]
    deg = deg_ref[...]

    means = []
    for g in range(3):
        asum = agg[2 * g] + agg[2 * g + 1]
        dsum = jnp.maximum(deg[:, 2 * g] + deg[:, 2 * g + 1], 1.0)
        means.append(asum / dsum[:, None])

    wpn = wpn_ref[...]; wps = wps_ref[...]; bpos = bpos_ref[...]
    wnn = wnn_ref[...]; wns = wns_ref[...]; bneg = bneg_ref[...]

    def gnn(mean, x, wn, ws, b):
        return jnp.tanh(jnp.dot(mean, wn, preferred_element_type=f32)
                        + jnp.dot(x, ws, preferred_element_type=f32) + b)

    h = [
        gnn(means[0], x3[0], wpn, wps, bpos),
        gnn(means[0], x3[0], wnn, wns, bneg),
        gnn(means[1], x3[1], wnn, wns, bneg),
        gnn(means[2], x3[2], wnn, wns, bneg),
    ]

    w1 = w1_ref[...]; b1 = b1_ref[...]
    w2 = w2_ref[...]; b2 = b2_ref[...]
    w3 = w3_ref[...]; b3 = b3_ref[...]
    outs = (o0_ref, o1_ref, o2_ref, o3_ref)
    for hi, o_ref in zip(h, outs):
        t1 = jnp.tanh(jnp.dot(hi, w1, preferred_element_type=f32) + b1)
        t2 = jnp.tanh(jnp.dot(t1, w2, preferred_element_type=f32) + b2)
        o_ref[...] = jnp.tanh(jnp.dot(t2, w3, preferred_element_type=f32) + b3)


def _dense(x3, agg6, deg6, wpn, wps, bpos, wnn, wns, bneg, w1, b1, w2, b2, w3, b3):
    n, d = x3.shape[1], x3.shape[2]
    bm = 1000
    grid = (n // bm,)
    d2 = w1.shape[1]
    do = w3.shape[1]

    const = lambda *shape: pl.BlockSpec(shape, lambda i: tuple(0 for _ in shape))
    out_shape = tuple(jax.ShapeDtypeStruct((n, do), jnp.float32) for _ in range(4))
    return pl.pallas_call(
        _dense_body,
        grid=grid,
        in_specs=[
            pl.BlockSpec((3, bm, d), lambda i: (0, i, 0)),
            pl.BlockSpec((6, bm, d), lambda i: (0, i, 0)),
            pl.BlockSpec((bm, 6), lambda i: (i, 0)),
            const(d, d), const(d, d), const(1, d),
            const(d, d), const(d, d), const(1, d),
            const(d, d2), const(1, d2),
            const(d2, d), const(1, d),
            const(d, do), const(1, do),
        ],
        out_specs=tuple(pl.BlockSpec((bm, do), lambda i: (i, 0)) for _ in range(4)),
        out_shape=out_shape,
    )(x3, agg6, deg6, wpn, wps, bpos.reshape(1, d), wnn, wns, bneg.reshape(1, d),
      w1, b1.reshape(1, d2), w2, b2.reshape(1, d), w3, b3.reshape(1, do))


def kernel(x_original, edge_index_original, x_pp, edge_index_pp, x_fm, edge_index_fm,
           W_pos_nbr, W_pos_self, b_pos, W_neg_nbr, W_neg_self, b_neg,
           W1, b1, W2, b2, W3, b3):
    n, d = x_original.shape
    n_pad = ((n + NS * C - 1) // (NS * C)) * (NS * C)

    xs = [x_original, x_pp, x_fm]
    srcs = [edge_index_original[0], edge_index_pp[0], edge_index_fm[0]]
    dsts = [edge_index_original[1], edge_index_pp[1], edge_index_fm[1]]

    agg_part, deg_part = _sc_aggregate(xs, srcs, dsts, n_pad)
    agg6 = agg_part[:, :, :n, :].reshape(3 * NC, n, d)
    deg6 = deg_part[:, :, :n].reshape(3 * NC, n).T
    x3 = jnp.stack(xs)

    outs = _dense(x3, agg6, deg6, W_pos_nbr, W_pos_self, b_pos,
                  W_neg_nbr, W_neg_self, b_neg, W1, b1, W2, b2, W3, b3)
    return tuple(outs)
